# baseline (speedup 1.0000x reference)
import functools

import jax
import jax.numpy as jnp
from jax import lax
from jax.experimental import pallas as pl
from jax.experimental.pallas import tpu as pltpu
from jax.experimental.pallas import tpu_sc as plsc

_B, _N, _T = 16, 128, 17
_D_IN, _H = 128, 128
_BIG = 100000.0
_NW = 32
_DPW = (_B * _N) // _NW


def _r_kernel(h_ref, x_ref, Wall_ref, r_ref):
    z = jnp.concatenate([x_ref[0], h_ref[0]], axis=1)
    r_ref[0] = jnp.dot(z, Wall_ref[...], preferred_element_type=jnp.float32)


def _finish_kernel(r_ref, M_ref, hasnb_ref, msgb_ref, o2_ref, ob_ref, h_ref):
    r = r_ref[0]
    agg = jnp.where(hasnb_ref[0] > 0.0,
                    jnp.maximum((r[:, :_H] + M_ref[0]) + msgb_ref[...], 0.0),
                    -_BIG)
    h_ref[0] = jnp.maximum(
        r[:, 2 * _H:3 * _H] +
        jnp.dot(agg, o2_ref[...], preferred_element_type=jnp.float32)
        + ob_ref[...], 0.0)


def _sc_agg_body(msg2_hbm, idx_hbm, ncv_hbm, out_hbm,
                 msg2_v, idx_v, ncv_v, m_v):
    wid = lax.axis_index("s") * 2 + lax.axis_index("c")
    b = wid // 2
    dst0 = (wid % 2) * _DPW
    pltpu.sync_copy(msg2_hbm.at[b], msg2_v)
    pltpu.sync_copy(idx_hbm.at[b, pl.ds(dst0, _DPW)], idx_v)
    pltpu.sync_copy(ncv_hbm.at[b, pl.ds(dst0, _DPW)], ncv_v)

    def do_group(g, carry):
        ncv16 = ncv_v[pl.ds(g * 16, 16)]
        for l in range(16):
            d = g * 16 + l
            nch = ncv16[l]

            def do_chunk(c, accs):
                idx16 = idx_v[d, pl.ds(c * 16, 16)]
                for l2 in range(16):
                    src = idx16[l2]
                    accs = tuple(
                        jnp.maximum(accs[j], msg2_v[src, pl.ds(j * 16, 16)])
                        for j in range(8))
                return accs

            accs = tuple(jnp.full((16,), -_BIG, jnp.float32) for _ in range(8))
            accs = lax.fori_loop(0, nch, do_chunk, accs)
            for j in range(8):
                m_v[d, pl.ds(j * 16, 16)] = accs[j]
        return carry

    lax.fori_loop(0, _DPW // 16, do_group, 0)
    pltpu.sync_copy(m_v, out_hbm.at[b, pl.ds(dst0, _DPW)])


def _sc_agg(msg2, idx, ncv):
    mesh = plsc.VectorSubcoreMesh(core_axis_name="c", subcore_axis_name="s")
    return pl.kernel(
        _sc_agg_body,
        mesh=mesh,
        out_type=jax.ShapeDtypeStruct((_B, _N, _H), jnp.float32),
        scratch_types=[
            pltpu.VMEM((_N, _H), jnp.float32),
            pltpu.VMEM((_DPW, _N), jnp.int32),
            pltpu.VMEM((_DPW,), jnp.int32),
            pltpu.VMEM((_DPW, _H), jnp.float32),
        ],
    )(msg2, idx, ncv)


def _enc_kernel(node_ref, encW_ref, encb_ref, x_ref):
    x_ref[0] = jnp.dot(node_ref[0], encW_ref[...],
                       preferred_element_type=jnp.float32) + encb_ref[...]


@jax.jit
def kernel(node_fts, adj, lengths, enc_W, enc_b, m1_W, m2_W, msg_b,
           o1_W, o2_W, o_b, dec_W, dec_b):
    W_all = jnp.concatenate(
        [m1_W, m2_W, o1_W, jnp.zeros_like(o1_W)], axis=1)
    full = lambda shape: pl.BlockSpec(shape, lambda i: (0,) * len(shape))
    bspec = lambda *shape: pl.BlockSpec((1,) + tuple(shape),
                                        lambda i: (i,) + (0,) * len(shape))

    deg = jnp.sum(adj, axis=2).astype(jnp.int32)
    order = jnp.argsort(-adj, axis=2, stable=True).astype(jnp.int32)
    hasnb = (deg > 0).astype(jnp.float32)[..., None]
    k_iota = jnp.arange(_N, dtype=jnp.int32)[None, None, :]
    padidx = jnp.where(k_iota < deg[:, :, None], order, order[:, :, :1])
    ncv = (deg + 15) // 16

    x_enc = pl.pallas_call(
        _enc_kernel, grid=(_B,),
        in_specs=[bspec(_N, _D_IN), full((_D_IN, _H)), full((1, _H))],
        out_specs=bspec(_N, _H),
        out_shape=jax.ShapeDtypeStruct((_B, _N, _H), jnp.float32),
    )(node_fts, enc_W, enc_b.reshape(1, _H))

    r_call = pl.pallas_call(
        _r_kernel, grid=(_B,),
        in_specs=[bspec(_N, _H), bspec(_N, _H), full((2 * _H, 4 * _H))],
        out_specs=bspec(_N, 4 * _H),
        out_shape=jax.ShapeDtypeStruct((_B, _N, 4 * _H), jnp.float32),
    )
    finish_call = pl.pallas_call(
        _finish_kernel, grid=(_B,),
        in_specs=[bspec(_N, 4 * _H), bspec(_N, _H), bspec(_N, 1),
                  full((1, _H)), full((_H, _H)), full((1, _H))],
        out_specs=bspec(_N, _H),
        out_shape=jax.ShapeDtypeStruct((_B, _N, _H), jnp.float32),
    )

    def one_step(h):
        r = r_call(h, x_enc, W_all)
        M = _sc_agg(r[:, :, _H:2 * _H], padidx, ncv)
        return finish_call(r, M, hasnb, msg_b.reshape(1, _H), o2_W,
                           o_b.reshape(1, _H))

    def decode(h):
        z = jnp.concatenate([x_enc, h], axis=-1)
        return (z @ dec_W + dec_b)[..., 0]

    hidden = one_step(jnp.zeros((_B, _N, _H), jnp.float32))
    out = decode(hidden)

    def body(carry, i):
        h, o = carry
        h_new = one_step(h)
        cand = decode(h_new)
        is_not_done = (lengths > i + 1).astype(jnp.float32)[:, None]
        o_new = is_not_done * cand + (1.0 - is_not_done) * o
        return (h_new, o_new), None

    (_, out), _ = lax.scan(body, (hidden, out), jnp.arange(_T - 2) + 1)
    return out


# speedup vs baseline: 7.4105x; 7.4105x over previous
import functools

import jax
import jax.numpy as jnp
from jax.experimental import pallas as pl
from jax.experimental.pallas import tpu as pltpu

_B, _N, _T = 16, 128, 17
_D_IN, _H = 128, 128
_BIG = 100000.0


def _mpnn_kernel(lengths_ref, node_ref, adj_ref, encW_ref, encb_ref,
                 Wall_ref, msgb_ref, o2_ref, ob_ref, decW_ref, decb_ref,
                 out_ref, bias_ref):
    b = pl.program_id(0)
    x = jnp.dot(node_ref[0], encW_ref[...],
                preferred_element_type=jnp.float32) + encb_ref[...]
    adj = adj_ref[0]
    hasnb = jnp.max(adj, axis=1, keepdims=True) > 0.0
    nsteps = jnp.maximum(lengths_ref[b] - 1, 1)

    def compute_r(h):
        z = jnp.concatenate([x, h], axis=1)
        return jnp.dot(z, Wall_ref[...], preferred_element_type=jnp.float32)

    def finish(r, M):
        agg = jnp.where(hasnb,
                        jnp.maximum((r[:, :_H] + M) + msgb_ref[...], 0.0),
                        -_BIG)
        return jnp.maximum(
            r[:, 2 * _H:3 * _H] +
            jnp.dot(agg, o2_ref[...],
                    preferred_element_type=jnp.float32) + ob_ref[...],
            0.0)

    r = compute_r(jnp.zeros((_N, _H), jnp.float32))
    msg2 = r[:, _H:2 * _H]
    M = jnp.full((_N, _H), -_BIG, dtype=jnp.float32)
    for s in range(_N):
        bl = (jax.lax.broadcast_in_dim(adj[:, s:s + 1], (_N, _H), (0, 1))
              * 2e30 - 1e30)
        bias_ref[s] = bl
        M = jnp.maximum(M, jnp.minimum(msg2[s:s + 1, :], bl))
    h = finish(r, M)

    def step(_, h):
        r = compute_r(h)
        msg2 = r[:, _H:2 * _H]
        M = jnp.full((_N, _H), -_BIG, dtype=jnp.float32)
        for s in range(_N):
            M = jnp.maximum(M, jnp.minimum(msg2[s:s + 1, :], bias_ref[s]))
        return finish(r, M)

    h = jax.lax.fori_loop(1, nsteps, step, h, unroll=False)
    z = jnp.concatenate([x, h], axis=1)
    out_ref[0] = (jnp.dot(z, decW_ref[...],
                          preferred_element_type=jnp.float32) + decb_ref[0, 0])


@jax.jit
def kernel(node_fts, adj, lengths, enc_W, enc_b, m1_W, m2_W, msg_b,
           o1_W, o2_W, o_b, dec_W, dec_b):
    W_all = jnp.concatenate(
        [m1_W, m2_W, o1_W, jnp.zeros_like(o1_W)], axis=1)
    grid = (_B,)
    full = lambda shape: pl.BlockSpec(shape, lambda i: (0,) * len(shape))
    out = pl.pallas_call(
        _mpnn_kernel,
        grid=grid,
        in_specs=[
            pl.BlockSpec(memory_space=pltpu.SMEM),
            pl.BlockSpec((1, _N, _D_IN), lambda i: (i, 0, 0)),
            pl.BlockSpec((1, _N, _N), lambda i: (i, 0, 0)),
            full((_D_IN, _H)),
            full((1, _H)),
            full((2 * _H, 4 * _H)),
            full((1, _H)),
            full((_H, _H)),
            full((1, _H)),
            full((2 * _H, 1)),
            full((1, 1)),
        ],
        out_specs=pl.BlockSpec((1, _N, 1), lambda i: (i, 0, 0)),
        out_shape=jax.ShapeDtypeStruct((_B, _N, 1), jnp.float32),
        scratch_shapes=[pltpu.VMEM((_N, _N, _H), jnp.float32)],
        compiler_params=pltpu.CompilerParams(
            dimension_semantics=("arbitrary",)),
    )(lengths, node_fts, adj, enc_W, enc_b.reshape(1, _H), W_all,
      msg_b.reshape(1, _H), o2_W, o_b.reshape(1, _H), dec_W,
      dec_b.reshape(1, 1))
    return out[:, :, 0]


# speedup vs baseline: 7.4200x; 1.0013x over previous
import functools

import jax
import jax.numpy as jnp
from jax.experimental import pallas as pl
from jax.experimental.pallas import tpu as pltpu

_B, _N, _T = 16, 128, 17
_D_IN, _H = 128, 128
_BIG = 100000.0


def _mpnn_kernel(lengths_ref, node_ref, adj_ref, encW_ref, encb_ref,
                 Wall_ref, msgb_ref, o2_ref, ob_ref, decW_ref, decb_ref,
                 out_ref, bias_ref):
    b = pl.program_id(0)
    x = jnp.dot(node_ref[0], encW_ref[...],
                preferred_element_type=jnp.float32) + encb_ref[...]
    adj = adj_ref[0]
    hasnb = jnp.max(adj, axis=1, keepdims=True) > 0.0
    nsteps = jnp.maximum(lengths_ref[b] - 1, 1)

    def compute_r(h):
        z = jnp.concatenate([x, h], axis=1)
        return jnp.dot(z, Wall_ref[...], preferred_element_type=jnp.float32)

    def finish(r, M):
        agg = jnp.where(hasnb,
                        jnp.maximum((r[:, :_H] + M) + msgb_ref[...], 0.0),
                        -_BIG)
        return jnp.maximum(
            r[:, 2 * _H:3 * _H] +
            jnp.dot(agg, o2_ref[...],
                    preferred_element_type=jnp.float32) + ob_ref[...],
            0.0)

    r = compute_r(jnp.zeros((_N, _H), jnp.float32))
    msg2 = r[:, _H:2 * _H]
    M = jnp.full((_N, _H), -_BIG, dtype=jnp.float32)
    for s in range(_N):
        bl = (jax.lax.broadcast_in_dim(adj[:, s:s + 1], (_N, _H), (0, 1))
              * 2e30 - 1e30)
        bias_ref[s] = bl
        M = jnp.maximum(M, jnp.minimum(msg2[s:s + 1, :], bl))
    h = finish(r, M)

    def step(_, h):
        r = compute_r(h)
        msg2 = r[:, _H:2 * _H]
        M = jnp.full((_N, _H), -_BIG, dtype=jnp.float32)
        for s in range(_N):
            M = jnp.maximum(M, jnp.minimum(msg2[s:s + 1, :], bias_ref[s]))
        return finish(r, M)

    h = jax.lax.fori_loop(1, nsteps, step, h, unroll=False)
    z = jnp.concatenate([x, h], axis=1)
    out_ref[0] = (jnp.dot(z, decW_ref[...],
                          preferred_element_type=jnp.float32) + decb_ref[0, 0])


@jax.jit
def kernel(node_fts, adj, lengths, enc_W, enc_b, m1_W, m2_W, msg_b,
           o1_W, o2_W, o_b, dec_W, dec_b):
    W_all = jnp.concatenate(
        [m1_W, m2_W, o1_W, jnp.zeros_like(o1_W)], axis=1)
    grid = (_B,)
    full = lambda shape: pl.BlockSpec(shape, lambda i: (0,) * len(shape))
    out = pl.pallas_call(
        _mpnn_kernel,
        grid=grid,
        in_specs=[
            pl.BlockSpec(memory_space=pltpu.SMEM),
            pl.BlockSpec((1, _N, _D_IN), lambda i: (i, 0, 0)),
            pl.BlockSpec((1, _N, _N), lambda i: (i, 0, 0)),
            full((_D_IN, _H)),
            full((1, _H)),
            full((2 * _H, 4 * _H)),
            full((1, _H)),
            full((_H, _H)),
            full((1, _H)),
            full((2 * _H, 1)),
            full((1, 1)),
        ],
        out_specs=pl.BlockSpec((1, _N, 1), lambda i: (i, 0, 0)),
        out_shape=jax.ShapeDtypeStruct((_B, _N, 1), jnp.float32),
        scratch_shapes=[pltpu.VMEM((_N, _N, _H), jnp.float32)],
        compiler_params=pltpu.CompilerParams(
            dimension_semantics=("parallel",)),
    )(lengths, node_fts, adj, enc_W, enc_b.reshape(1, _H), W_all,
      msg_b.reshape(1, _H), o2_W, o_b.reshape(1, _H), dec_W,
      dec_b.reshape(1, 1))
    return out[:, :, 0]
